# baseline (speedup 1.0000x reference)
import functools

import jax
import jax.numpy as jnp
from jax import lax
from jax.experimental import pallas as pl
from jax.experimental.pallas import tpu as pltpu
from jax.experimental.pallas import tpu_sc as plsc

_EPS = 1e-12
_CHUNK = 160
_GRPS = _CHUNK // 16


def _sc_fused(table, idx_flat, tt_flat, posc2, gamma, beta, n_rows, d):
    info = plsc.get_sparse_core_info()
    nc, ns = info.num_cores, info.num_subcores
    nw = nc * ns
    rows_per_w = n_rows // nw
    n_chunks = rows_per_w // _CHUNK
    n_pairs = n_chunks // 2
    mesh = plsc.VectorSubcoreMesh(core_axis_name="c", subcore_axis_name="s")

    @functools.partial(
        pl.kernel,
        mesh=mesh,
        compiler_params=pltpu.CompilerParams(needs_layout_passes=False),
        out_type=jax.ShapeDtypeStruct((n_rows, d), jnp.float32),
        scratch_types=[
            pltpu.VMEM((rows_per_w,), jnp.int32),
            pltpu.VMEM((rows_per_w,), jnp.int32),
            pltpu.VMEM((400, d), jnp.float32),
            pltpu.VMEM((d,), jnp.float32),
            pltpu.VMEM((d,), jnp.float32),
            pltpu.VMEM((_CHUNK, d), jnp.float32),
            pltpu.VMEM((_CHUNK, d), jnp.float32),
            pltpu.VMEM((272,), jnp.float32),
            pltpu.VMEM((272,), jnp.float32),
            pltpu.SemaphoreType.DMA,
            pltpu.SemaphoreType.DMA,
            pltpu.SemaphoreType.DMA,
            pltpu.SemaphoreType.DMA,
        ],
    )
    def k(table_hbm, idx_hbm, tt_hbm, posc2_hbm, gamma_hbm, beta_hbm,
          out_hbm, idx_v, tt_v, posc2_v, gamma_v, beta_v, rows_a, rows_b,
          s_scr, q_scr, gsem_a, gsem_b, ssem_a, ssem_b):
        wid = lax.axis_index("s") * nc + lax.axis_index("c")
        base = wid * rows_per_w

        pltpu.sync_copy(idx_hbm.at[pl.ds(base, rows_per_w)], idx_v)
        pltpu.sync_copy(tt_hbm.at[pl.ds(base, rows_per_w)], tt_v)
        pltpu.sync_copy(posc2_hbm, posc2_v)
        pltpu.sync_copy(gamma_hbm, gamma_v)
        pltpu.sync_copy(beta_hbm, beta_v)

        def start_gather(c, buf, sem):
            pltpu.async_copy(
                table_hbm.at[idx_v.at[pl.ds(c * _CHUNK, _CHUNK)]], buf, sem)

        def start_scatter(c, buf, sem):
            pltpu.async_copy(
                buf, out_hbm.at[pl.ds(base + c * _CHUNK, _CHUNK)], sem)

        def wait_gather(buf, sem):
            pltpu.make_async_copy(
                out_hbm.at[pl.ds(base, _CHUNK)], buf, sem).wait()

        def wait_scatter(buf, sem):
            pltpu.make_async_copy(
                buf, out_hbm.at[pl.ds(base, _CHUNK)], sem).wait()

        def ln_chunk(c, rows_v):
            def group_body(g, carry):
                tok0 = g * 16
                goff = c * _CHUNK + tok0
                tvec = tt_v[pl.ds(goff, 16)]
                xcols = []
                for kk in range(16):
                    tok = tok0 + kk
                    t = tvec[kk]
                    p = lax.rem(goff + kk, 200)
                    row = p + 200 * t
                    acc_s = None
                    acc_q = None
                    for j in range(8):
                        xw = rows_v[tok, pl.ds(16 * j, 16)]
                        xp = posc2_v[row, pl.ds(16 * j, 16)]
                        x = xw + xp
                        rows_v[tok, pl.ds(16 * j, 16)] = x
                        acc_s = x if acc_s is None else acc_s + x
                        acc_q = x * x if acc_q is None else acc_q + x * x
                    s_scr[pl.ds(kk * 17, 16)] = acc_s
                    q_scr[pl.ds(kk * 17, 16)] = acc_q
                i16 = lax.iota(jnp.int32, 16)
                i17 = i16 * 17
                sumv = None
                sqv = None
                for l in range(16):
                    gs = plsc.load_gather(s_scr, [i17 + l])
                    gq = plsc.load_gather(q_scr, [i17 + l])
                    sumv = gs if sumv is None else sumv + gs
                    sqv = gq if sqv is None else sqv + gq
                meanv = sumv * (1.0 / 128.0)
                varv = sqv * (1.0 / 128.0) - meanv * meanv + _EPS
                u = plsc.bitcast(varv, jnp.int32)
                u = 0x5F3759DF - lax.shift_right_arithmetic(u, 1)
                y = plsc.bitcast(u, jnp.float32)
                for _ in range(3):
                    y = y * (1.5 - 0.5 * varv * y * y)
                av = y
                bv = meanv * y
                for kk in range(16):
                    tok = tok0 + kk
                    a = av[kk]
                    b = bv[kk]
                    for j in range(8):
                        gj = gamma_v[pl.ds(16 * j, 16)]
                        bj = beta_v[pl.ds(16 * j, 16)]
                        x = rows_v[tok, pl.ds(16 * j, 16)]
                        rows_v[tok, pl.ds(16 * j, 16)] = (x * a - b) * gj + bj
                return carry

            lax.fori_loop(0, _GRPS, group_body, 0)

        start_gather(0, rows_a, gsem_a)

        def pair_body(i, carry):
            ca = 2 * i
            cb = 2 * i + 1

            @pl.when(i > 0)
            def _():
                wait_scatter(rows_b, ssem_b)

            start_gather(cb, rows_b, gsem_b)
            wait_gather(rows_a, gsem_a)
            ln_chunk(ca, rows_a)
            start_scatter(ca, rows_a, ssem_a)
            wait_gather(rows_b, gsem_b)
            ln_chunk(cb, rows_b)
            wait_scatter(rows_a, ssem_a)

            @pl.when(i < n_pairs - 1)
            def _():
                start_gather(ca + 2, rows_a, gsem_a)

            start_scatter(cb, rows_b, ssem_b)
            return carry

        lax.fori_loop(0, n_pairs, pair_body, 0)
        wait_scatter(rows_b, ssem_b)

    return k(table, idx_flat, tt_flat, posc2, gamma, beta)


def kernel(input_ids, token_type_ids, word_emb, pos_emb, type_emb, gamma, beta):
    b, s = input_ids.shape
    d = word_emb.shape[1]
    posc2 = jnp.concatenate(
        [pos_emb[:s] + type_emb[0][None, :],
         pos_emb[:s] + type_emb[1][None, :]], axis=0)
    out = _sc_fused(
        word_emb,
        input_ids.reshape(-1).astype(jnp.int32),
        token_type_ids.reshape(-1).astype(jnp.int32),
        posc2,
        gamma,
        beta,
        b * s,
        d,
    )
    return out.reshape(b, s, d)


# speedup vs baseline: 7.0259x; 7.0259x over previous
import functools

import jax
import jax.numpy as jnp
from jax import lax
from jax.experimental import pallas as pl
from jax.experimental.pallas import tpu as pltpu
from jax.experimental.pallas import tpu_sc as plsc

_EPS = 1e-12
_N_SLICES = 4
_B_BLK = 32


def _sc_gather(table, idx_flat, n_rows, d, chunk):
    info = plsc.get_sparse_core_info()
    nc, ns = info.num_cores, info.num_subcores
    nw = nc * ns
    rows_per_w = n_rows // nw
    n_chunks = rows_per_w // chunk
    mesh = plsc.VectorSubcoreMesh(core_axis_name="c", subcore_axis_name="s")

    @functools.partial(
        pl.kernel,
        mesh=mesh,
        out_type=jax.ShapeDtypeStruct((n_rows, d), jnp.float32),
        scratch_types=[
            pltpu.VMEM((chunk,), jnp.int32),
            pltpu.VMEM((chunk, d), jnp.float32),
            pltpu.SemaphoreType.DMA,
        ],
    )
    def gather_kernel(table_hbm, idx_hbm, out_hbm, idx_v, rows_v, sem):
        wid = lax.axis_index("s") * nc + lax.axis_index("c")
        base = wid * rows_per_w

        def body(i, carry):
            off = base + i * chunk
            pltpu.sync_copy(idx_hbm.at[pl.ds(off, chunk)], idx_v)
            pltpu.async_copy(table_hbm.at[idx_v], rows_v, sem).wait()
            pltpu.sync_copy(rows_v, out_hbm.at[pl.ds(off, chunk)])
            return carry

        lax.fori_loop(0, n_chunks, body, 0)

    return gather_kernel(table, idx_flat)


def _tc_ln_body(g_ref, tt_ref, posc_ref, delta_ref, gamma_ref,
                beta_ref, o_ref):
    x = g_ref[...]
    t = tt_ref[...].astype(jnp.float32)[..., None]
    x = x + posc_ref[...][None, :, :] + t * delta_ref[...][None, :, :]
    mean = jnp.mean(x, axis=-1, keepdims=True)
    xc = x - mean
    var = jnp.mean(xc * xc, axis=-1, keepdims=True)
    y = xc * lax.rsqrt(var + _EPS)
    o_ref[...] = y * gamma_ref[...][None, :, :] + beta_ref[...][None, :, :]


def _tc_ln_slice(out_buf, gathered_k, tt_k, posc, delta, gamma2, beta2,
                 slice_base, b, s, d):
    b_slice = gathered_k.shape[0]
    grid = (b_slice // _B_BLK,)
    blk0 = slice_base // _B_BLK
    out_spec = pl.BlockSpec((_B_BLK, s, d), lambda i: (blk0 + i, 0, 0))
    in_specs = [
        pl.BlockSpec((_B_BLK, s, d), lambda i: (i, 0, 0)),
        pl.BlockSpec((_B_BLK, s), lambda i: (i, 0)),
        pl.BlockSpec((s, d), lambda i: (0, 0)),
        pl.BlockSpec((1, d), lambda i: (0, 0)),
        pl.BlockSpec((1, d), lambda i: (0, 0)),
        pl.BlockSpec((1, d), lambda i: (0, 0)),
    ]
    args = (gathered_k, tt_k, posc, delta, gamma2, beta2)
    if out_buf is None:
        body = _tc_ln_body
        aliases = {}
    else:
        body = lambda o_in, *rest: _tc_ln_body(*rest)
        in_specs = [out_spec] + in_specs
        args = (out_buf,) + args
        aliases = {0: 0}
    return pl.pallas_call(
        body,
        grid=grid,
        in_specs=in_specs,
        out_specs=out_spec,
        out_shape=jax.ShapeDtypeStruct((b, s, d), jnp.float32),
        input_output_aliases=aliases,
    )(*args)


def kernel(input_ids, token_type_ids, word_emb, pos_emb, type_emb, gamma, beta):
    b, s = input_ids.shape
    d = word_emb.shape[1]
    posc = pos_emb[:s] + type_emb[0][None, :]
    delta = (type_emb[1] - type_emb[0])[None, :]
    gamma2 = gamma[None, :]
    beta2 = beta[None, :]
    ids32 = input_ids.astype(jnp.int32)
    tt32 = token_type_ids.astype(jnp.int32)

    b_slice = b // _N_SLICES
    gathered = [
        _sc_gather(
            word_emb,
            ids32[k * b_slice:(k + 1) * b_slice].reshape(-1),
            b_slice * s,
            d,
            chunk=800,
        ).reshape(b_slice, s, d)
        for k in range(_N_SLICES)
    ]

    out = None
    for k in range(_N_SLICES):
        out = _tc_ln_slice(
            out,
            gathered[k],
            tt32[k * b_slice:(k + 1) * b_slice],
            posc,
            delta,
            gamma2,
            beta2,
            slice_base=k * b_slice,
            b=b, s=s, d=d,
        )
    return out


# speedup vs baseline: 7.2008x; 1.0249x over previous
import functools

import jax
import jax.numpy as jnp
from jax import lax
from jax.experimental import pallas as pl
from jax.experimental.pallas import tpu as pltpu
from jax.experimental.pallas import tpu_sc as plsc

_EPS = 1e-12
_N_SLICES = 4
_B_BLK = 64


def _sc_gather(table, idx_flat, n_rows, d, chunk):
    info = plsc.get_sparse_core_info()
    nc, ns = info.num_cores, info.num_subcores
    nw = nc * ns
    rows_per_w = n_rows // nw
    n_chunks = rows_per_w // chunk
    mesh = plsc.VectorSubcoreMesh(core_axis_name="c", subcore_axis_name="s")

    @functools.partial(
        pl.kernel,
        mesh=mesh,
        out_type=jax.ShapeDtypeStruct((n_rows, d), jnp.float32),
        scratch_types=[
            pltpu.VMEM((chunk,), jnp.int32),
            pltpu.VMEM((chunk, d), jnp.float32),
            pltpu.SemaphoreType.DMA,
        ],
    )
    def gather_kernel(table_hbm, idx_hbm, out_hbm, idx_v, rows_v, sem):
        wid = lax.axis_index("s") * nc + lax.axis_index("c")
        base = wid * rows_per_w

        def body(i, carry):
            off = base + i * chunk
            pltpu.sync_copy(idx_hbm.at[pl.ds(off, chunk)], idx_v)
            pltpu.async_copy(table_hbm.at[idx_v], rows_v, sem).wait()
            pltpu.sync_copy(rows_v, out_hbm.at[pl.ds(off, chunk)])
            return carry

        lax.fori_loop(0, n_chunks, body, 0)

    return gather_kernel(table, idx_flat)


def _tc_ln_body(g_ref, tt_ref, posc_ref, delta_ref, gamma_ref,
                beta_ref, o_ref):
    x = g_ref[...]
    t = tt_ref[...].astype(jnp.float32)[..., None]
    x = x + posc_ref[...][None, :, :] + t * delta_ref[...][None, :, :]
    mean = jnp.mean(x, axis=-1, keepdims=True)
    xc = x - mean
    var = jnp.mean(xc * xc, axis=-1, keepdims=True)
    y = xc * lax.rsqrt(var + _EPS)
    o_ref[...] = y * gamma_ref[...][None, :, :] + beta_ref[...][None, :, :]


def _tc_ln_slice(out_buf, gathered_k, tt_k, posc, delta, gamma2, beta2,
                 slice_base, b, s, d):
    b_slice = gathered_k.shape[0]
    grid = (b_slice // _B_BLK,)
    blk0 = slice_base // _B_BLK
    out_spec = pl.BlockSpec((_B_BLK, s, d), lambda i: (blk0 + i, 0, 0))
    in_specs = [
        pl.BlockSpec((_B_BLK, s, d), lambda i: (i, 0, 0)),
        pl.BlockSpec((_B_BLK, s), lambda i: (i, 0)),
        pl.BlockSpec((s, d), lambda i: (0, 0)),
        pl.BlockSpec((1, d), lambda i: (0, 0)),
        pl.BlockSpec((1, d), lambda i: (0, 0)),
        pl.BlockSpec((1, d), lambda i: (0, 0)),
    ]
    args = (gathered_k, tt_k, posc, delta, gamma2, beta2)
    if out_buf is None:
        body = _tc_ln_body
        aliases = {}
    else:
        body = lambda o_in, *rest: _tc_ln_body(*rest)
        in_specs = [out_spec] + in_specs
        args = (out_buf,) + args
        aliases = {0: 0}
    return pl.pallas_call(
        body,
        grid=grid,
        in_specs=in_specs,
        out_specs=out_spec,
        out_shape=jax.ShapeDtypeStruct((b, s, d), jnp.float32),
        input_output_aliases=aliases,
    )(*args)


def kernel(input_ids, token_type_ids, word_emb, pos_emb, type_emb, gamma, beta):
    b, s = input_ids.shape
    d = word_emb.shape[1]
    posc = pos_emb[:s] + type_emb[0][None, :]
    delta = (type_emb[1] - type_emb[0])[None, :]
    gamma2 = gamma[None, :]
    beta2 = beta[None, :]
    ids32 = input_ids.astype(jnp.int32)
    tt32 = token_type_ids.astype(jnp.int32)

    b_slice = b // _N_SLICES
    gathered = [
        _sc_gather(
            word_emb,
            ids32[k * b_slice:(k + 1) * b_slice].reshape(-1),
            b_slice * s,
            d,
            chunk=800,
        ).reshape(b_slice, s, d)
        for k in range(_N_SLICES)
    ]

    out = None
    for k in range(_N_SLICES):
        out = _tc_ln_slice(
            out,
            gathered[k],
            tt32[k * b_slice:(k + 1) * b_slice],
            posc,
            delta,
            gamma2,
            beta2,
            slice_base=k * b_slice,
            b=b, s=s, d=d,
        )
    return out
